# baseline (speedup 1.0000x reference)
import functools

import jax
import jax.numpy as jnp
from jax import lax
from jax.experimental import pallas as pl
from jax.experimental.pallas import tpu as pltpu
from jax.experimental.pallas import tpu_sc as plsc

N_NODES = 10000
NPAD = 10240
N_EDGES = 320000
D = 128
NC = 2
NS = 16
NW = NC * NS
CHUNK = 128
EPT = 10240
NCHUNK = EPT // CHUNK
IG = 8
NGRP = NCHUNK // IG
EPAD = NW * EPT
RPT = NPAD // NS
DEG_W = 16


@functools.cache
def _mesh():
    return plsc.VectorSubcoreMesh(
        core_axis_name="c", subcore_axis_name="s", num_cores=NC, num_subcores=NS
    )


def _deg_body(dst_hbm, ones_hbm, zerosf_hbm, out_hbm, idx_v, ones_v, acc_sh):
    c = lax.axis_index("c")
    s = lax.axis_index("s")
    t = c * NS + s
    pltpu.sync_copy(dst_hbm.at[t], idx_v)
    pltpu.sync_copy(ones_hbm, ones_v)
    pltpu.sync_copy(
        zerosf_hbm.at[pl.ds(s * RPT, RPT)], acc_sh.at[pl.ds(s * RPT, RPT)]
    )
    plsc.subcore_barrier()

    def step(g, carry):
        for k in range(IG):
            pltpu.sync_copy(ones_v, acc_sh.at[idx_v.at[g, k]], add=True)
        return carry

    lax.fori_loop(0, NGRP, step, 0)
    plsc.subcore_barrier()
    pltpu.sync_copy(
        acc_sh.at[pl.ds(s * RPT, RPT)],
        out_hbm.at[c, pl.ds(s * RPT, RPT)],
    )


def _msg_body(y_hbm, src_hbm, dst_hbm, out_hbm, sidx, didx, buf0, buf1,
              acc_sh, semi0, semi1, semd0, semd1, sems0, sems1):
    c = lax.axis_index("c")
    s = lax.axis_index("s")
    t = c * NS + s
    bufs = (buf0, buf1)
    semi = (semi0, semi1)
    semd = (semd0, semd1)
    sems = (sems0, sems1)

    def idx_load(g, a):
        pltpu.async_copy(src_hbm.at[t, g], sidx.at[a], semi[a])
        pltpu.async_copy(dst_hbm.at[t, g], didx.at[a], semi[a])

    def idx_wait(g, a):
        pltpu.make_async_copy(src_hbm.at[t, g], sidx.at[a], semi[a]).wait()
        pltpu.make_async_copy(dst_hbm.at[t, g], didx.at[a], semi[a]).wait()

    def gather(a, k, b):
        pltpu.async_copy(y_hbm.at[sidx.at[a, k]], bufs[b], semd[b])

    def gather_wait(a, k, b):
        pltpu.make_async_copy(y_hbm.at[sidx.at[a, k]], bufs[b],
                              semd[b]).wait()

    def scatter(a, k, b):
        pltpu.async_copy(bufs[b], acc_sh.at[didx.at[a, k]], sems[b],
                         add=True)

    idx_load(0, 0)
    idx_load(1, 1)
    pltpu.sync_copy(
        y_hbm.at[pl.ds(s * RPT, RPT)],
        acc_sh.at[pl.ds(s * RPT, RPT)],
    )
    plsc.subcore_barrier()
    idx_wait(0, 0)
    gather(0, 0, 0)

    def scatter_drain(b):
        pltpu.make_async_copy(bufs[b], acc_sh.at[didx.at[0, 0]],
                              sems[b]).wait()

    def group_pair(gp, carry):
        for a in range(2):
            g = gp * 2 + a

            @pl.when(g + 1 < NGRP)
            def _():
                idx_wait(g + 1, 1 - a)

            for k in range(IG):
                b = k % 2
                gather_wait(a, k, b)
                scatter(a, k, b)
                if k >= 1:
                    scatter_drain(1 - b)
                if k + 1 < IG:
                    gather(a, k + 1, 1 - b)
                else:
                    @pl.when(g + 1 < NGRP)
                    def _():
                        gather(1 - a, 0, 1 - b)

            scatter_drain(1)

            @pl.when(g + 2 < NGRP)
            def _():
                idx_load(g + 2, a)

        return carry

    lax.fori_loop(0, NGRP // 2, group_pair, 0)
    plsc.subcore_barrier()
    pltpu.sync_copy(
        acc_sh.at[pl.ds(s * RPT, RPT)],
        out_hbm.at[c, pl.ds(s * RPT, RPT)],
    )


@functools.cache
def _deg_kernel():
    return pl.kernel(
        _deg_body,
        out_type=jax.ShapeDtypeStruct((NC, NPAD), jnp.float32),
        mesh=_mesh(),
        scratch_types=[
            pltpu.VMEM((NGRP, IG, CHUNK), jnp.int32),
            pltpu.VMEM((CHUNK,), jnp.float32),
            pltpu.VMEM_SHARED((NPAD,), jnp.float32),
        ],
    )


@functools.cache
def _msg_kernel():
    return pl.kernel(
        _msg_body,
        out_type=jax.ShapeDtypeStruct((NC, NPAD, D), jnp.float32),
        mesh=_mesh(),
        scratch_types=[
            pltpu.VMEM((2, IG, CHUNK), jnp.int32),
            pltpu.VMEM((2, IG, CHUNK), jnp.int32),
            pltpu.VMEM((CHUNK, D), jnp.float32),
            pltpu.VMEM((CHUNK, D), jnp.float32),
            pltpu.VMEM_SHARED((NPAD, D), jnp.float32),
            pltpu.SemaphoreType.DMA,
            pltpu.SemaphoreType.DMA,
            pltpu.SemaphoreType.DMA,
            pltpu.SemaphoreType.DMA,
            pltpu.SemaphoreType.DMA,
            pltpu.SemaphoreType.DMA,
        ],
    )


BM = 2048


def _dis(dp_ref):
    return lax.rsqrt(dp_ref[0, :, 0:1] + dp_ref[1, :, 0:1] + 1.0)


def _scale_mm_body(x_ref, w_ref, dp_ref, o_ref):
    xw = jnp.dot(
        x_ref[...], w_ref[...],
        preferred_element_type=jnp.float32, precision=lax.Precision.HIGHEST,
    )
    o_ref[...] = xw * _dis(dp_ref)


def _mid_body(p_ref, y_ref, b_ref, w_ref, dp_ref, o_ref):
    dis = _dis(dp_ref)
    agg = p_ref[0] + p_ref[1] - y_ref[...]
    h = jnp.maximum(agg * dis + b_ref[...], 0.0)
    hw = jnp.dot(
        h, w_ref[...],
        preferred_element_type=jnp.float32, precision=lax.Precision.HIGHEST,
    )
    o_ref[...] = hw * dis


def _final_body(p_ref, y_ref, b_ref, w_ref, b2_ref, dp_ref, o_ref):
    dis = _dis(dp_ref)
    agg = p_ref[0] + p_ref[1] - y_ref[...]
    h = jnp.maximum(agg * dis + b_ref[...], 0.0)
    hw = jnp.dot(
        h, w_ref[...],
        preferred_element_type=jnp.float32, precision=lax.Precision.HIGHEST,
    )
    o_ref[...] = hw + b2_ref[...]


def _row_spec():
    return pl.BlockSpec((BM, D), lambda i: (i, 0))


def _full_spec(shape):
    return pl.BlockSpec(shape, lambda i: tuple(0 for _ in shape))


def _p_spec():
    return pl.BlockSpec((NC, BM, D), lambda i: (0, i, 0))


def _dp_spec():
    return pl.BlockSpec((NC, BM, 1), lambda i: (0, i, 0))


_GRID = NPAD // BM

_scale_mm = pl.pallas_call(
    _scale_mm_body,
    grid=(_GRID,),
    in_specs=[_row_spec(), _full_spec((D, D)), _dp_spec()],
    out_specs=_row_spec(),
    out_shape=jax.ShapeDtypeStruct((NPAD, D), jnp.float32),
)

_mid = pl.pallas_call(
    _mid_body,
    grid=(_GRID,),
    in_specs=[_p_spec(), _row_spec(), _full_spec((1, D)), _full_spec((D, D)),
              _dp_spec()],
    out_specs=_row_spec(),
    out_shape=jax.ShapeDtypeStruct((NPAD, D), jnp.float32),
)

_final = pl.pallas_call(
    _final_body,
    grid=(_GRID,),
    in_specs=[_p_spec(), _row_spec(), _full_spec((1, D)), _full_spec((D, D)),
              _full_spec((1, D)), _dp_spec()],
    out_specs=_row_spec(),
    out_shape=jax.ShapeDtypeStruct((NPAD, D), jnp.float32),
)


@jax.jit
def kernel(x, edge_index, W1, b1, W2, b2, Wfc, bfc):
    ei = edge_index.astype(jnp.int32)
    npe = EPAD - N_EDGES
    src_pad = jnp.arange(npe, dtype=jnp.int32) % N_NODES
    dst_pad = N_NODES + (jnp.arange(npe, dtype=jnp.int32) % (NPAD - N_NODES))
    src4 = jnp.concatenate([ei[0], src_pad]).reshape(NW, NGRP, IG, CHUNK)
    dst4 = jnp.concatenate([ei[1], dst_pad]).reshape(NW, NGRP, IG, CHUNK)
    zerosf = jnp.zeros((NPAD,), jnp.float32)
    ones1 = jnp.ones((CHUNK,), jnp.float32)
    x_pad = jnp.pad(x, ((0, NPAD - N_NODES), (0, 0)))

    dp = _deg_kernel()(dst4, ones1, zerosf).reshape(NC, NPAD, 1)

    y1 = _scale_mm(x_pad, W1, dp)
    p1 = _msg_kernel()(y1, src4, dst4)
    y2 = _mid(p1, y1, b1.reshape(1, D), W2, dp)
    p2 = _msg_kernel()(y2, src4, dst4)
    out = _final(p2, y2, b2.reshape(1, D), Wfc, bfc.reshape(1, D), dp)
    return out[:N_NODES]


# speedup vs baseline: 1.1076x; 1.1076x over previous
import functools

import jax
import jax.numpy as jnp
from jax import lax
from jax.experimental import pallas as pl
from jax.experimental.pallas import tpu as pltpu
from jax.experimental.pallas import tpu_sc as plsc

N_NODES = 10000
NPAD = 10240
N_EDGES = 320000
D = 128
NC = 2
NS = 16
NW = NC * NS
CHUNK = 128
EPT = 10240
NCHUNK = EPT // CHUNK
IG = 8
NGRP = NCHUNK // IG
EPAD = NW * EPT
RPT = NPAD // NS
DEG_W = 16


@functools.cache
def _mesh():
    return plsc.VectorSubcoreMesh(
        core_axis_name="c", subcore_axis_name="s", num_cores=NC, num_subcores=NS
    )


def _deg_body(dst_hbm, ones_hbm, zerosf_hbm, out_hbm, idx_v, ones_v, acc_sh):
    c = lax.axis_index("c")
    s = lax.axis_index("s")
    t = c * NS + s
    pltpu.sync_copy(dst_hbm.at[t], idx_v)
    pltpu.sync_copy(ones_hbm, ones_v)
    pltpu.sync_copy(
        zerosf_hbm.at[pl.ds(s * RPT, RPT)], acc_sh.at[pl.ds(s * RPT, RPT)]
    )
    plsc.subcore_barrier()

    def step(g, carry):
        for k in range(IG):
            pltpu.sync_copy(ones_v, acc_sh.at[idx_v.at[g, k]], add=True)
        return carry

    lax.fori_loop(0, NGRP, step, 0)
    plsc.subcore_barrier()
    pltpu.sync_copy(
        acc_sh.at[pl.ds(s * RPT, RPT)],
        out_hbm.at[c, pl.ds(s * RPT, RPT)],
    )


def _msg_body(y_hbm, src_hbm, dst_hbm, out_hbm, sidx, didx, buf0, buf1,
              acc_sh, semi0, semi1, semd0, semd1):
    c = lax.axis_index("c")
    s = lax.axis_index("s")
    t = c * NS + s
    bufs = (buf0, buf1)
    semi = (semi0, semi1)
    semd = (semd0, semd1)

    def idx_load(g, a):
        pltpu.async_copy(src_hbm.at[t, g], sidx.at[a], semi[a])
        pltpu.async_copy(dst_hbm.at[t, g], didx.at[a], semi[a])

    def idx_wait(g, a):
        pltpu.make_async_copy(src_hbm.at[t, g], sidx.at[a], semi[a]).wait()
        pltpu.make_async_copy(dst_hbm.at[t, g], didx.at[a], semi[a]).wait()

    def gather(a, k, b):
        pltpu.async_copy(y_hbm.at[sidx.at[a, k]], bufs[b], semd[b])

    def gather_wait(a, k, b):
        pltpu.make_async_copy(y_hbm.at[sidx.at[a, k]], bufs[b],
                              semd[b]).wait()

    idx_load(0, 0)
    idx_load(1, 1)
    pltpu.sync_copy(
        y_hbm.at[pl.ds(s * RPT, RPT)],
        acc_sh.at[pl.ds(s * RPT, RPT)],
    )
    plsc.subcore_barrier()
    idx_wait(0, 0)
    gather(0, 0, 0)

    def group_pair(gp, carry):
        for a in range(2):
            g = gp * 2 + a

            @pl.when(g + 1 < NGRP)
            def _():
                idx_wait(g + 1, 1 - a)

            for k in range(IG):
                b = k % 2
                if k + 1 < IG:
                    gather(a, k + 1, 1 - b)
                else:
                    @pl.when(g + 1 < NGRP)
                    def _():
                        gather(1 - a, 0, 1 - b)

                gather_wait(a, k, b)
                pltpu.sync_copy(bufs[b], acc_sh.at[didx.at[a, k]], add=True)

            @pl.when(g + 2 < NGRP)
            def _():
                idx_load(g + 2, a)

        return carry

    lax.fori_loop(0, NGRP // 2, group_pair, 0)
    plsc.subcore_barrier()
    pltpu.sync_copy(
        acc_sh.at[pl.ds(s * RPT, RPT)],
        out_hbm.at[c, pl.ds(s * RPT, RPT)],
    )


@functools.cache
def _deg_kernel():
    return pl.kernel(
        _deg_body,
        out_type=jax.ShapeDtypeStruct((NC, NPAD), jnp.float32),
        mesh=_mesh(),
        scratch_types=[
            pltpu.VMEM((NGRP, IG, CHUNK), jnp.int32),
            pltpu.VMEM((CHUNK,), jnp.float32),
            pltpu.VMEM_SHARED((NPAD,), jnp.float32),
        ],
    )


@functools.cache
def _msg_kernel():
    return pl.kernel(
        _msg_body,
        out_type=jax.ShapeDtypeStruct((NC, NPAD, D), jnp.float32),
        mesh=_mesh(),
        scratch_types=[
            pltpu.VMEM((2, IG, CHUNK), jnp.int32),
            pltpu.VMEM((2, IG, CHUNK), jnp.int32),
            pltpu.VMEM((CHUNK, D), jnp.float32),
            pltpu.VMEM((CHUNK, D), jnp.float32),
            pltpu.VMEM_SHARED((NPAD, D), jnp.float32),
            pltpu.SemaphoreType.DMA,
            pltpu.SemaphoreType.DMA,
            pltpu.SemaphoreType.DMA,
            pltpu.SemaphoreType.DMA,
        ],
    )


BM = 2048


def _dis(dp_ref):
    return lax.rsqrt(dp_ref[0, :, 0:1] + dp_ref[1, :, 0:1] + 1.0)


def _scale_mm_body(x_ref, w_ref, dp_ref, o_ref):
    xw = jnp.dot(
        x_ref[...], w_ref[...],
        preferred_element_type=jnp.float32, precision=lax.Precision.HIGHEST,
    )
    o_ref[...] = xw * _dis(dp_ref)


def _mid_body(p_ref, y_ref, b_ref, w_ref, dp_ref, o_ref):
    dis = _dis(dp_ref)
    agg = p_ref[0] + p_ref[1] - y_ref[...]
    h = jnp.maximum(agg * dis + b_ref[...], 0.0)
    hw = jnp.dot(
        h, w_ref[...],
        preferred_element_type=jnp.float32, precision=lax.Precision.HIGHEST,
    )
    o_ref[...] = hw * dis


def _final_body(p_ref, y_ref, b_ref, w_ref, b2_ref, dp_ref, o_ref):
    dis = _dis(dp_ref)
    agg = p_ref[0] + p_ref[1] - y_ref[...]
    h = jnp.maximum(agg * dis + b_ref[...], 0.0)
    hw = jnp.dot(
        h, w_ref[...],
        preferred_element_type=jnp.float32, precision=lax.Precision.HIGHEST,
    )
    o_ref[...] = hw + b2_ref[...]


def _row_spec():
    return pl.BlockSpec((BM, D), lambda i: (i, 0))


def _full_spec(shape):
    return pl.BlockSpec(shape, lambda i: tuple(0 for _ in shape))


def _p_spec():
    return pl.BlockSpec((NC, BM, D), lambda i: (0, i, 0))


def _dp_spec():
    return pl.BlockSpec((NC, BM, 1), lambda i: (0, i, 0))


_GRID = NPAD // BM

_scale_mm = pl.pallas_call(
    _scale_mm_body,
    grid=(_GRID,),
    in_specs=[_row_spec(), _full_spec((D, D)), _dp_spec()],
    out_specs=_row_spec(),
    out_shape=jax.ShapeDtypeStruct((NPAD, D), jnp.float32),
)

_mid = pl.pallas_call(
    _mid_body,
    grid=(_GRID,),
    in_specs=[_p_spec(), _row_spec(), _full_spec((1, D)), _full_spec((D, D)),
              _dp_spec()],
    out_specs=_row_spec(),
    out_shape=jax.ShapeDtypeStruct((NPAD, D), jnp.float32),
)

_final = pl.pallas_call(
    _final_body,
    grid=(_GRID,),
    in_specs=[_p_spec(), _row_spec(), _full_spec((1, D)), _full_spec((D, D)),
              _full_spec((1, D)), _dp_spec()],
    out_specs=_row_spec(),
    out_shape=jax.ShapeDtypeStruct((NPAD, D), jnp.float32),
)


@jax.jit
def kernel(x, edge_index, W1, b1, W2, b2, Wfc, bfc):
    ei = edge_index.astype(jnp.int32)
    npe = EPAD - N_EDGES
    src_pad = jnp.arange(npe, dtype=jnp.int32) % N_NODES
    dst_pad = N_NODES + (jnp.arange(npe, dtype=jnp.int32) % (NPAD - N_NODES))
    src4 = jnp.concatenate([ei[0], src_pad]).reshape(NW, NGRP, IG, CHUNK)
    dst4 = jnp.concatenate([ei[1], dst_pad]).reshape(NW, NGRP, IG, CHUNK)
    zerosf = jnp.zeros((NPAD,), jnp.float32)
    ones1 = jnp.ones((CHUNK,), jnp.float32)
    x_pad = jnp.pad(x, ((0, NPAD - N_NODES), (0, 0)))

    dp = _deg_kernel()(dst4, ones1, zerosf).reshape(NC, NPAD, 1)

    y1 = _scale_mm(x_pad, W1, dp)
    p1 = _msg_kernel()(y1, src4, dst4)
    y2 = _mid(p1, y1, b1.reshape(1, D), W2, dp)
    p2 = _msg_kernel()(y2, src4, dst4)
    out = _final(p2, y2, b2.reshape(1, D), Wfc, bfc.reshape(1, D), dp)
    return out[:N_NODES]
